# baseline (speedup 1.0000x reference)
import functools

import jax
import jax.numpy as jnp
from jax import lax
from jax.experimental import pallas as pl
from jax.experimental.pallas import tpu as pltpu
from jax.experimental.pallas import tpu_sc as plsc

_NUM_CATEGORIES = 1000
_USER_DIM = 64
_MOVIE_DIM = 32
_CAT_DIM = 32
_BATCH = 16384
_HIST = 20
_MARGIN = 0.1

_NC = 2
_NS = 16
_NW = _NC * _NS
_BPW = _BATCH // _NW
_CHUNK = 128
_NCHUNK = _BPW // _CHUNK
_L = 16
_NBLK = _BPW // _L


@functools.partial(
    pl.kernel,
    out_type=jax.ShapeDtypeStruct((_BATCH,), jnp.float32),
    mesh=plsc.VectorSubcoreMesh(core_axis_name="c", subcore_axis_name="s",
                                num_cores=_NC, num_subcores=_NS),
    compiler_params=pltpu.CompilerParams(needs_layout_passes=False,
                                         use_tc_tiling_on_sc=False),
    scratch_types=[
        pltpu.VMEM((_NCHUNK, _CHUNK), jnp.int32),
        pltpu.VMEM((_NCHUNK, _CHUNK), jnp.int32),
        pltpu.VMEM((_BPW * _HIST,), jnp.int32),
        pltpu.VMEM((_USER_DIM, _BPW), jnp.float32),
        pltpu.VMEM((_MOVIE_DIM, _BPW), jnp.float32),
        pltpu.VMEM((_BPW,), jnp.float32),
        pltpu.VMEM((_BPW,), jnp.float32),
        pltpu.VMEM((_NUM_CATEGORIES, _CAT_DIM), jnp.float32),
        pltpu.VMEM((_BPW,), jnp.float32),
        pltpu.SemaphoreType.DMA,
    ],
)
def _sc_kernel(uid_hbm, mid_hbm, cats_hbm, euT_hbm, bu_hbm, emT_hbm, ec_hbm,
               bm_hbm, out_hbm,
               uid_v, mid_v, cats_v, uT_v, mT_v, ub_v, mb_v, ctab_v,
               out_v, sem):
    wid = lax.axis_index("s") * _NC + lax.axis_index("c")
    base = wid * _BPW

    for k in range(_NCHUNK):
        pltpu.sync_copy(uid_hbm.at[pl.ds(base + k * _CHUNK, _CHUNK)],
                        uid_v.at[k])
        pltpu.sync_copy(mid_hbm.at[pl.ds(base + k * _CHUNK, _CHUNK)],
                        mid_v.at[k])
    pltpu.sync_copy(cats_hbm.at[pl.ds(base * _HIST, _BPW * _HIST)], cats_v)
    pltpu.sync_copy(ec_hbm, ctab_v)

    copies = []
    for k in range(_NCHUNK):
        dst = pl.ds(k * _CHUNK, _CHUNK)
        for d in range(_USER_DIM):
            copies.append(pltpu.async_copy(
                euT_hbm.at[d].at[uid_v.at[k]], uT_v.at[d, dst], sem))
        for d in range(_MOVIE_DIM):
            copies.append(pltpu.async_copy(
                emT_hbm.at[d].at[mid_v.at[k]], mT_v.at[d, dst], sem))
        copies.append(pltpu.async_copy(bu_hbm.at[uid_v.at[k]],
                                       ub_v.at[dst], sem))
        copies.append(pltpu.async_copy(bm_hbm.at[mid_v.at[k]],
                                       mb_v.at[dst], sem))
    for c in copies:
        c.wait()

    lanes = lax.iota(jnp.int32, _L)

    def step(t, carry):
        elems = t * _L + lanes
        cat_base = elems * _HIST

        cvecs = [plsc.load_gather(cats_v, [cat_base + j])
                 for j in range(_HIST)]
        cnt = (cvecs[0] != 0).astype(jnp.float32)
        for j in range(1, _HIST):
            cnt = cnt + (cvecs[j] != 0).astype(jnp.float32)
        inv = 1.0 / jnp.maximum(cnt, 1.0)

        sl = pl.ds(t * _L, _L)
        acc = ub_v[sl] + mb_v[sl]

        for d in range(_MOVIE_DIM):
            acc = acc + uT_v[d, sl] * mT_v[d, sl]

        for dd in range(_CAT_DIM):
            col = (dd + lanes) & (_CAT_DIM - 1)
            u_hi = plsc.load_gather(uT_v, [col + _MOVIE_DIM, elems])
            s = plsc.load_gather(ctab_v, [cvecs[0], col])
            for j in range(1, _HIST):
                s = s + plsc.load_gather(ctab_v, [cvecs[j], col])
            acc = acc + u_hi * (s * inv)

        prob = 1.0 / (1.0 + jnp.exp(-acc))
        out_v[sl] = prob * (1.0 + 2.0 * _MARGIN) - _MARGIN
        return carry

    lax.fori_loop(0, _NBLK, step, 0)
    pltpu.sync_copy(out_v, out_hbm.at[pl.ds(base, _BPW)])


def kernel(user_id, movie_id, movie_categories, emb_users, bias_user,
           emb_movies, emb_movie_cats, bias_movie):
    cats_flat = movie_categories.reshape(-1)
    bu_flat = bias_user.reshape(-1)
    bm_flat = bias_movie.reshape(-1)
    return _sc_kernel(user_id, movie_id, cats_flat, emb_users.T, bu_flat,
                      emb_movies.T, emb_movie_cats, bm_flat)


# speedup vs baseline: 6.3740x; 6.3740x over previous
import functools

import jax
import jax.numpy as jnp
from jax import lax
from jax.experimental import pallas as pl
from jax.experimental.pallas import tpu as pltpu
from jax.experimental.pallas import tpu_sc as plsc

_NUM_CATEGORIES = 1000
_USER_DIM = 64
_MOVIE_DIM = 32
_CAT_DIM = 32
_BATCH = 16384
_HIST = 20
_MARGIN = 0.1

_NC = 2
_NS = 16
_NW = _NC * _NS
_BPW = _BATCH // _NW
_CHUNK = 128
_NCHUNK = _BPW // _CHUNK
_L = 16
_HALF = _BPW // 2
_NBLK_H = _HALF // _L

_mesh = plsc.VectorSubcoreMesh(core_axis_name="c", subcore_axis_name="s",
                               num_cores=_NC, num_subcores=_NS)


@functools.partial(
    pl.kernel,
    out_type=(jax.ShapeDtypeStruct((_BATCH, 128), jnp.float32),
              jax.ShapeDtypeStruct((_BATCH, 128), jnp.float32)),
    mesh=_mesh,
    compiler_params=pltpu.CompilerParams(needs_layout_passes=False,
                                         use_tc_tiling_on_sc=True),
    scratch_types=[
        pltpu.VMEM((_NCHUNK, _CHUNK), jnp.int32),
        pltpu.VMEM((_BPW, 128), jnp.float32),
        pltpu.SemaphoreType.DMA,
    ],
)
def _gatherer(uidp_hbm, midq_hbm, eup_hbm, emq_hbm, outu_hbm, outm_hbm,
              idx_v, rows_v, sem):
    wid = lax.axis_index("s") * _NC + lax.axis_index("c")
    base = wid * _BPW

    def gather_table(ids_hbm, tab_hbm, out_hbm):
        for k in range(_NCHUNK):
            pltpu.sync_copy(ids_hbm.at[pl.ds(base + k * _CHUNK, _CHUNK)],
                            idx_v.at[k])
        cps = [pltpu.async_copy(tab_hbm.at[idx_v.at[k]],
                                rows_v.at[pl.ds(k * _CHUNK, _CHUNK)], sem)
               for k in range(_NCHUNK)]
        for c in cps:
            c.wait()
        pltpu.sync_copy(rows_v, out_hbm.at[pl.ds(base, _BPW)])

    gather_table(uidp_hbm, eup_hbm, outu_hbm)
    gather_table(midq_hbm, emq_hbm, outm_hbm)


@functools.partial(
    pl.kernel,
    out_type=jax.ShapeDtypeStruct((_BATCH,), jnp.float32),
    mesh=_mesh,
    compiler_params=pltpu.CompilerParams(needs_layout_passes=False,
                                         use_tc_tiling_on_sc=False),
    scratch_types=[
        pltpu.VMEM((_NCHUNK, _CHUNK), jnp.int32),
        pltpu.VMEM((_NCHUNK, _CHUNK), jnp.int32),
        pltpu.VMEM((_BPW,), jnp.int32),
        pltpu.VMEM((_BPW,), jnp.int32),
        pltpu.VMEM((_BPW * _HIST,), jnp.int32),
        pltpu.VMEM((_HALF, 128), jnp.float32),
        pltpu.VMEM((_HALF, 128), jnp.float32),
        pltpu.VMEM((_BPW,), jnp.float32),
        pltpu.VMEM((_BPW,), jnp.float32),
        pltpu.VMEM((_NUM_CATEGORIES, _CAT_DIM), jnp.float32),
        pltpu.VMEM((_BPW,), jnp.float32),
        pltpu.SemaphoreType.DMA,
    ],
)
def _scorer(uid_hbm, mid_hbm, cats_hbm, rowsu_hbm, rowsm_hbm, bu_hbm,
            bm_hbm, ec_hbm, out_hbm,
            uid_v, mid_v, uid1_v, mid1_v, cats_v, urows_v, mrows_v,
            ub_v, mb_v, ctab_v, out_v, sem):
    wid = lax.axis_index("s") * _NC + lax.axis_index("c")
    base = wid * _BPW

    for k in range(_NCHUNK):
        pltpu.sync_copy(uid_hbm.at[pl.ds(base + k * _CHUNK, _CHUNK)],
                        uid_v.at[k])
        pltpu.sync_copy(mid_hbm.at[pl.ds(base + k * _CHUNK, _CHUNK)],
                        mid_v.at[k])
    pltpu.sync_copy(uid_hbm.at[pl.ds(base, _BPW)], uid1_v)
    pltpu.sync_copy(mid_hbm.at[pl.ds(base, _BPW)], mid1_v)
    pltpu.sync_copy(cats_hbm.at[pl.ds(base * _HIST, _BPW * _HIST)], cats_v)
    pltpu.sync_copy(ec_hbm, ctab_v)

    copies = []
    for k in range(_NCHUNK):
        dst = pl.ds(k * _CHUNK, _CHUNK)
        copies.append(pltpu.async_copy(bu_hbm.at[uid_v.at[k]],
                                       ub_v.at[dst], sem))
        copies.append(pltpu.async_copy(bm_hbm.at[mid_v.at[k]],
                                       mb_v.at[dst], sem))
    for c in copies:
        c.wait()

    lanes = lax.iota(jnp.int32, _L)

    def step(t, carry):
            off = t * _L
            @pl.when(t == _NBLK_H)
            def _():
                hsl = pl.ds(base + _HALF, _HALF)
                pltpu.sync_copy(rowsu_hbm.at[hsl], urows_v)
                pltpu.sync_copy(rowsm_hbm.at[hsl], mrows_v)

            elems = (t & (_NBLK_H - 1)) * _L + lanes
            goff = off + lanes
            cat_base = goff * _HIST

            uid16 = uid1_v[pl.ds(off, _L)]
            mid16 = mid1_v[pl.ds(off, _L)]
            ucol0 = (uid16 & 1) * _USER_DIM
            mcol0 = (mid16 & 3) * _MOVIE_DIM

            cvecs = [plsc.load_gather(cats_v, [cat_base + jj])
                     for jj in range(_HIST)]
            cnt = (cvecs[0] != 0).astype(jnp.float32)
            for jj in range(1, _HIST):
                cnt = cnt + (cvecs[jj] != 0).astype(jnp.float32)
            inv = 1.0 / jnp.maximum(cnt, 1.0)

            sl = pl.ds(off, _L)
            acc = ub_v[sl] + mb_v[sl]

            for dd in range(_CAT_DIM):
                col = (dd + lanes) & (_CAT_DIM - 1)
                u_lo = plsc.load_gather(urows_v, [elems, ucol0 + col])
                m_d = plsc.load_gather(mrows_v, [elems, mcol0 + col])
                acc = acc + u_lo * m_d
                u_hi = plsc.load_gather(urows_v,
                                        [elems, ucol0 + _MOVIE_DIM + col])
                s = plsc.load_gather(ctab_v, [cvecs[0], col])
                for jj in range(1, _HIST):
                    s = s + plsc.load_gather(ctab_v, [cvecs[jj], col])
                acc = acc + u_hi * (s * inv)

            prob = 1.0 / (1.0 + jnp.exp(-acc))
            out_v[sl] = prob * (1.0 + 2.0 * _MARGIN) - _MARGIN
            return carry

    hsl0 = pl.ds(base, _HALF)
    pltpu.sync_copy(rowsu_hbm.at[hsl0], urows_v)
    pltpu.sync_copy(rowsm_hbm.at[hsl0], mrows_v)
    lax.fori_loop(0, 2 * _NBLK_H, step, 0)

    pltpu.sync_copy(out_v, out_hbm.at[pl.ds(base, _BPW)])


def kernel(user_id, movie_id, movie_categories, emb_users, bias_user,
           emb_movies, emb_movie_cats, bias_movie):
    eu_p = emb_users.reshape(500000, 128)
    em_q = emb_movies.reshape(250000, 128)
    rows_u, rows_m = _gatherer(user_id >> 1, movie_id >> 2, eu_p, em_q)
    cats_flat = movie_categories.reshape(-1)
    bu_flat = bias_user.reshape(-1)
    bm_flat = bias_movie.reshape(-1)
    return _scorer(user_id, movie_id, cats_flat, rows_u, rows_m, bu_flat,
                   bm_flat, emb_movie_cats)


# speedup vs baseline: 7.1681x; 1.1246x over previous
import functools

import jax
import jax.numpy as jnp
from jax import lax
from jax.experimental import pallas as pl
from jax.experimental.pallas import tpu as pltpu
from jax.experimental.pallas import tpu_sc as plsc

_NUM_CATEGORIES = 1000
_USER_DIM = 64
_MOVIE_DIM = 32
_CAT_DIM = 32
_BATCH = 16384
_HIST = 20
_MARGIN = 0.1

_NC = 2
_NS = 16
_NW = _NC * _NS
_BPW = _BATCH // _NW
_CHUNK = 128
_NCHUNK = _BPW // _CHUNK
_L = 16
_NBLK = _BPW // _L


@functools.partial(
    pl.kernel,
    out_type=jax.ShapeDtypeStruct((_BATCH,), jnp.float32),
    mesh=plsc.VectorSubcoreMesh(core_axis_name="c", subcore_axis_name="s",
                                num_cores=_NC, num_subcores=_NS),
    compiler_params=pltpu.CompilerParams(needs_layout_passes=False,
                                         use_tc_tiling_on_sc=False),
    scratch_types=[
        pltpu.VMEM((_NCHUNK, _CHUNK), jnp.int32),
        pltpu.VMEM((_NCHUNK, _CHUNK), jnp.int32),
        pltpu.VMEM((_BPW * _HIST,), jnp.int32),
        pltpu.VMEM((_BPW, _USER_DIM), jnp.float32),
        pltpu.VMEM((_BPW, _MOVIE_DIM), jnp.float32),
        pltpu.VMEM((_BPW,), jnp.float32),
        pltpu.VMEM((_BPW,), jnp.float32),
        pltpu.VMEM((_NUM_CATEGORIES, _CAT_DIM), jnp.float32),
        pltpu.VMEM((_BPW,), jnp.float32),
        pltpu.SemaphoreType.DMA,
    ],
)
def _sc_kernel(uid_hbm, mid_hbm, cats_hbm, eu_hbm, bu_hbm, em_hbm, ec_hbm,
               bm_hbm, out_hbm,
               uid_v, mid_v, cats_v, urows_v, mrows_v, ub_v, mb_v, ctab_v,
               out_v, sem):
    wid = lax.axis_index("s") * _NC + lax.axis_index("c")
    base = wid * _BPW

    for k in range(_NCHUNK):
        pltpu.sync_copy(uid_hbm.at[pl.ds(base + k * _CHUNK, _CHUNK)],
                        uid_v.at[k])
        pltpu.sync_copy(mid_hbm.at[pl.ds(base + k * _CHUNK, _CHUNK)],
                        mid_v.at[k])
    pltpu.sync_copy(cats_hbm.at[pl.ds(base * _HIST, _BPW * _HIST)], cats_v)
    pltpu.sync_copy(ec_hbm, ctab_v)

    copies = []
    for k in range(_NCHUNK):
        dst = pl.ds(k * _CHUNK, _CHUNK)
        copies.append(pltpu.async_copy(eu_hbm.at[uid_v.at[k]],
                                       urows_v.at[dst], sem))
        copies.append(pltpu.async_copy(em_hbm.at[mid_v.at[k]],
                                       mrows_v.at[dst], sem))
        copies.append(pltpu.async_copy(bu_hbm.at[uid_v.at[k]],
                                       ub_v.at[dst], sem))
        copies.append(pltpu.async_copy(bm_hbm.at[mid_v.at[k]],
                                       mb_v.at[dst], sem))
    for c in copies:
        c.wait()

    lanes = lax.iota(jnp.int32, _L)

    def step(t, carry):
        rows = t * _L + lanes
        cat_base = rows * _HIST

        cvecs = [plsc.load_gather(cats_v, [cat_base + j])
                 for j in range(_HIST)]
        cnt = (cvecs[0] != 0).astype(jnp.float32)
        for j in range(1, _HIST):
            cnt = cnt + (cvecs[j] != 0).astype(jnp.float32)
        inv = 1.0 / jnp.maximum(cnt, 1.0)

        acc = ub_v[pl.ds(t * _L, _L)] + mb_v[pl.ds(t * _L, _L)]

        acc2 = jnp.zeros((_L,), jnp.float32)
        for dd in range(_CAT_DIM):
            col = (dd + lanes) & (_CAT_DIM - 1)
            u_lo = plsc.load_gather(urows_v, [rows, col])
            m_d = plsc.load_gather(mrows_v, [rows, col])
            acc = acc + u_lo * m_d
            u_hi = plsc.load_gather(urows_v, [rows, col + _MOVIE_DIM])
            terms = [plsc.load_gather(ctab_v, [cvecs[j], col])
                     for j in range(_HIST)]
            while len(terms) > 1:
                nxt = [terms[i] + terms[i + 1]
                       for i in range(0, len(terms) - 1, 2)]
                if len(terms) % 2:
                    nxt.append(terms[-1])
                terms = nxt
            acc2 = acc2 + u_hi * (terms[0] * inv)
        acc = acc + acc2

        prob = 1.0 / (1.0 + jnp.exp(-acc))
        out_v[pl.ds(t * _L, _L)] = prob * (1.0 + 2.0 * _MARGIN) - _MARGIN
        return carry

    lax.fori_loop(0, _NBLK, step, 0)
    pltpu.sync_copy(out_v, out_hbm.at[pl.ds(base, _BPW)])


def kernel(user_id, movie_id, movie_categories, emb_users, bias_user,
           emb_movies, emb_movie_cats, bias_movie):
    cats_flat = movie_categories.reshape(-1)
    bu_flat = bias_user.reshape(-1)
    bm_flat = bias_movie.reshape(-1)
    return _sc_kernel(user_id, movie_id, cats_flat, emb_users, bu_flat,
                      emb_movies, emb_movie_cats, bm_flat)


# speedup vs baseline: 7.2575x; 1.0125x over previous
import functools

import jax
import jax.numpy as jnp
from jax import lax
from jax.experimental import pallas as pl
from jax.experimental.pallas import tpu as pltpu
from jax.experimental.pallas import tpu_sc as plsc

_NUM_CATEGORIES = 1000
_USER_DIM = 64
_MOVIE_DIM = 32
_CAT_DIM = 32
_BATCH = 16384
_HIST = 20
_MARGIN = 0.1

_NC = 2
_NS = 16
_NW = _NC * _NS
_BPW = _BATCH // _NW
_CHUNK = 128
_NCHUNK = _BPW // _CHUNK
_L = 16
_NBLK = _BPW // _L


@functools.partial(
    pl.kernel,
    out_type=jax.ShapeDtypeStruct((_BATCH,), jnp.float32),
    mesh=plsc.VectorSubcoreMesh(core_axis_name="c", subcore_axis_name="s",
                                num_cores=_NC, num_subcores=_NS),
    compiler_params=pltpu.CompilerParams(needs_layout_passes=False,
                                         use_tc_tiling_on_sc=False),
    scratch_types=[
        pltpu.VMEM((_NCHUNK, _CHUNK), jnp.int32),
        pltpu.VMEM((_NCHUNK, _CHUNK), jnp.int32),
        pltpu.VMEM((_BPW * _HIST,), jnp.int32),
        pltpu.VMEM((_BPW, _USER_DIM), jnp.float32),
        pltpu.VMEM((_BPW, _MOVIE_DIM), jnp.float32),
        pltpu.VMEM((_BPW,), jnp.float32),
        pltpu.VMEM((_BPW,), jnp.float32),
        pltpu.VMEM((_NUM_CATEGORIES, _CAT_DIM), jnp.float32),
        pltpu.VMEM((_BPW,), jnp.float32),
        pltpu.SemaphoreType.DMA,
    ],
)
def _sc_kernel(uid_hbm, mid_hbm, cats_hbm, eu_hbm, bu_hbm, em_hbm, ec_hbm,
               bm_hbm, out_hbm,
               uid_v, mid_v, cats_v, urows_v, mrows_v, ub_v, mb_v, ctab_v,
               out_v, sem):
    wid = lax.axis_index("s") * _NC + lax.axis_index("c")
    base = wid * _BPW

    for k in range(_NCHUNK):
        pltpu.sync_copy(uid_hbm.at[pl.ds(base + k * _CHUNK, _CHUNK)],
                        uid_v.at[k])
        pltpu.sync_copy(mid_hbm.at[pl.ds(base + k * _CHUNK, _CHUNK)],
                        mid_v.at[k])
    pltpu.sync_copy(cats_hbm.at[pl.ds(base * _HIST, _BPW * _HIST)], cats_v)
    pltpu.sync_copy(ec_hbm, ctab_v)

    copies = []
    for k in range(_NCHUNK):
        dst = pl.ds(k * _CHUNK, _CHUNK)
        copies.append(pltpu.async_copy(eu_hbm.at[uid_v.at[k]],
                                       urows_v.at[dst], sem))
        copies.append(pltpu.async_copy(em_hbm.at[mid_v.at[k]],
                                       mrows_v.at[dst], sem))
        copies.append(pltpu.async_copy(bu_hbm.at[uid_v.at[k]],
                                       ub_v.at[dst], sem))
        copies.append(pltpu.async_copy(bm_hbm.at[mid_v.at[k]],
                                       mb_v.at[dst], sem))
    for c in copies:
        c.wait()

    lanes = lax.iota(jnp.int32, _L)

    @plsc.parallel_loop(0, _NBLK)
    def step(t):
        rows = t * _L + lanes
        cat_base = rows * _HIST

        cvecs = [plsc.load_gather(cats_v, [cat_base + j])
                 for j in range(_HIST)]
        cnt = (cvecs[0] != 0).astype(jnp.float32)
        for j in range(1, _HIST):
            cnt = cnt + (cvecs[j] != 0).astype(jnp.float32)
        inv = 1.0 / jnp.maximum(cnt, 1.0)

        acc = ub_v[pl.ds(t * _L, _L)] + mb_v[pl.ds(t * _L, _L)]

        for dd in range(_CAT_DIM):
            col = (dd + lanes) & (_CAT_DIM - 1)
            u_lo = plsc.load_gather(urows_v, [rows, col])
            m_d = plsc.load_gather(mrows_v, [rows, col])
            acc = acc + u_lo * m_d
            u_hi = plsc.load_gather(urows_v, [rows, col + _MOVIE_DIM])
            s = plsc.load_gather(ctab_v, [cvecs[0], col])
            for j in range(1, _HIST):
                s = s + plsc.load_gather(ctab_v, [cvecs[j], col])
            acc = acc + u_hi * (s * inv)

        prob = 1.0 / (1.0 + jnp.exp(-acc))
        out_v[pl.ds(t * _L, _L)] = prob * (1.0 + 2.0 * _MARGIN) - _MARGIN

    pltpu.sync_copy(out_v, out_hbm.at[pl.ds(base, _BPW)])


def kernel(user_id, movie_id, movie_categories, emb_users, bias_user,
           emb_movies, emb_movie_cats, bias_movie):
    cats_flat = movie_categories.reshape(-1)
    bu_flat = bias_user.reshape(-1)
    bm_flat = bias_movie.reshape(-1)
    return _sc_kernel(user_id, movie_id, cats_flat, emb_users, bu_flat,
                      emb_movies, emb_movie_cats, bm_flat)
